# baseline (speedup 1.0000x reference)
import functools

import jax
import jax.numpy as jnp
from jax import lax
from jax.experimental import pallas as pl
from jax.experimental.pallas import tpu as pltpu
from jax.experimental.pallas import tpu_sc as plsc

EMBED = 64
NUM_CORES = 2
NUM_SUBCORES = 16
NW = NUM_CORES * NUM_SUBCORES
BLK = 128
TPAD = 129


def _sc_gather_t(idx_flat, table2):
    b_total = idx_flat.shape[0]
    n_blocks = b_total // BLK
    blocks_per_w = n_blocks // NW
    b_per_w = b_total // NW
    mesh = plsc.VectorSubcoreMesh(core_axis_name="c", subcore_axis_name="s")

    @functools.partial(
        pl.kernel,
        out_type=jax.ShapeDtypeStruct((n_blocks * 8 * 8, 128), jnp.float32),
        mesh=mesh,
        scratch_types=[
            pltpu.VMEM((b_per_w,), jnp.int32),
            pltpu.VMEM((BLK, EMBED), jnp.float32),
            pltpu.VMEM((BLK, EMBED), jnp.float32),
            pltpu.VMEM((EMBED, TPAD), jnp.float32),
            pltpu.VMEM((EMBED, TPAD), jnp.float32),
            pltpu.SemaphoreType.DMA,
            pltpu.SemaphoreType.DMA,
            pltpu.SemaphoreType.DMA,
            pltpu.SemaphoreType.DMA,
        ],
        compiler_params=pltpu.CompilerParams(
            use_tc_tiling_on_sc=False, needs_layout_passes=False),
    )
    def k(idx_hbm, table_hbm, out_hbm, idx_v, rows0, rows1, t0, t1,
          gsem0, gsem1, osem0, osem1):
        wid = lax.axis_index("s") * NUM_CORES + lax.axis_index("c")
        base = wid * b_per_w
        rows = (rows0, rows1)
        tb = (t0, t1)
        gsem = (gsem0, gsem1)
        osem = (osem0, osem1)

        pltpu.sync_copy(idx_hbm.at[pl.ds(base, b_per_w)], idx_v)
        lanes = lax.iota(jnp.int32, 16)
        zeros16 = jnp.zeros((16,), jnp.int32)

        def gather(j, b):
            pltpu.async_copy(
                table_hbm.at[idx_v.at[pl.ds(j * BLK, BLK)]],
                rows[b], gsem[b])

        def stage(j, b):
            pltpu.make_async_copy(
                table_hbm.at[idx_v.at[pl.ds(0, BLK)]],
                rows[b], gsem[b]).wait()

            @plsc.parallel_loop(0, BLK, step=1, unroll=8)
            def _(bi):
                bvec = jnp.full((16,), bi, jnp.int32)
                for j16 in range(EMBED // 16):
                    vals = rows[b][bi, pl.ds(j16 * 16, 16)]
                    plsc.store_scatter(
                        tb[b], [j16 * 16 + lanes, bvec], vals)
            blk = wid * blocks_per_w + j
            h = blk // 32
            bc = blk - h * 32
            for er in range(8):
                pltpu.async_copy(
                    tb[b].at[pl.ds(er * 8, 8), pl.ds(0, 128)],
                    out_hbm.at[pl.ds(((h * 8 + er) * 32 + bc) * 8, 8)],
                    osem[b])

        def wait_writes(b):
            pltpu.make_async_copy(
                tb[b].at[pl.ds(0, EMBED), pl.ds(0, 128)],
                out_hbm.at[pl.ds(0, EMBED)], osem[b]).wait()

        gather(0, 0)

        def body(p, carry):
            for b in range(2):
                j = p * 2 + b
                if b == 0:
                    gather(j + 1, 1)
                else:
                    @pl.when(j + 1 < blocks_per_w)
                    def _():
                        gather(j + 1, 0)

                @pl.when(p >= 1)
                def _():
                    wait_writes(b)

                stage(j, b)
            return carry

        lax.fori_loop(0, blocks_per_w // 2, body, 0)
        wait_writes(0)
        wait_writes(1)

    return k(idx_flat, table2)


VPAD = 129


def _sc_table_prep(tabT, tail_rows):
    vocab = tabT.shape[1]
    full_stripes = vocab // 128
    spw = (full_stripes // (2 * NW)) * 2
    leftover = full_stripes - spw * NW
    tail_v0 = full_stripes * 128
    mesh = plsc.VectorSubcoreMesh(core_axis_name="c", subcore_axis_name="s")

    @functools.partial(
        pl.kernel,
        out_type=jax.ShapeDtypeStruct((vocab, 128), jnp.float32),
        mesh=mesh,
        scratch_types=[
            pltpu.VMEM((EMBED, 128), jnp.float32),
            pltpu.VMEM((EMBED, 128), jnp.float32),
            pltpu.VMEM((EMBED, EMBED), jnp.float32),
            pltpu.VMEM((128, VPAD), jnp.float32),
            pltpu.VMEM((128, VPAD), jnp.float32),
            pltpu.SemaphoreType.DMA,
            pltpu.SemaphoreType.DMA,
            pltpu.SemaphoreType.DMA,
            pltpu.SemaphoreType.DMA,
        ],
        compiler_params=pltpu.CompilerParams(
            use_tc_tiling_on_sc=True, needs_layout_passes=False),
    )
    def ka(tabT_hbm, tail_hbm, tpad_hbm, buf0, buf1, buft, tb0, tb1,
           gsem0, gsem1, osem0, osem1):
        wid = lax.axis_index("s") * NUM_CORES + lax.axis_index("c")
        sbase = wid * spw
        buf = (buf0, buf1)
        tbf = (tb0, tb1)
        gsem = (gsem0, gsem1)
        osem = (osem0, osem1)
        lanes = lax.iota(jnp.int32, 16)

        def read(i, b):
            pltpu.async_copy(
                tabT_hbm.at[:, pl.ds((sbase + i) * 128, 128)],
                buf[b], gsem[b])

        def transpose(b, nv16, src_ref=None):
            sref = buf[b] if src_ref is None else src_ref

            @plsc.parallel_loop(0, EMBED, step=1, unroll=8)
            def _(e):
                evec = jnp.full((16,), e, jnp.int32)
                for v16 in range(nv16):
                    vals = sref[e, pl.ds(v16 * 16, 16)]
                    plsc.store_scatter(
                        tbf[b], [v16 * 16 + lanes, evec], vals)

        def stage(i, b):
            pltpu.make_async_copy(
                tabT_hbm.at[:, pl.ds(0, 128)], buf[b], gsem[b]).wait()
            transpose(b, 8)
            pltpu.async_copy(
                tbf[b].at[:, pl.ds(0, 128)],
                tpad_hbm.at[pl.ds((sbase + i) * 128, 128), :],
                osem[b])

        def wait_writes(b):
            pltpu.make_async_copy(
                tbf[b].at[:, pl.ds(0, 128)],
                tpad_hbm.at[pl.ds(0, 128), :], osem[b]).wait()

        read(0, 0)

        def body(p, carry):
            for b in range(2):
                i = p * 2 + b
                if b == 0:
                    read(i + 1, 1)
                else:
                    @pl.when(i + 1 < spw)
                    def _():
                        read(i + 1, 0)

                @pl.when(p >= 1)
                def _():
                    wait_writes(b)

                stage(i, b)
            return carry

        lax.fori_loop(0, spw // 2, body, 0)
        wait_writes(0)
        wait_writes(1)

        @pl.when(wid < leftover)
        def _():
            s = full_stripes - leftover + wid
            pltpu.sync_copy(tabT_hbm.at[:, pl.ds(s * 128, 128)], buf0)
            transpose(0, 8)
            pltpu.async_copy(
                tbf[0].at[:, pl.ds(0, 128)],
                tpad_hbm.at[pl.ds(s * 128, 128), :], osem0)
            wait_writes(0)

        @pl.when(wid == leftover)
        def _():
            pltpu.sync_copy(tail_hbm, buft)
            transpose(0, 4, src_ref=buft)
            pltpu.async_copy(
                tbf[0].at[pl.ds(0, 64), pl.ds(0, 128)],
                tpad_hbm.at[pl.ds(tail_v0, 64), :], osem0)
            pltpu.make_async_copy(
                tbf[0].at[pl.ds(0, 64), pl.ds(0, 128)],
                tpad_hbm.at[pl.ds(0, 64), :], osem0).wait()

    return ka(tabT, tail_rows)


def kernel(input, table):
    vocab = table.shape[0]
    n_tail = vocab % 128
    tailT = lax.slice_in_dim(table.T, vocab - n_tail, vocab, axis=1)
    tpad = _sc_table_prep(table.T, tailT)
    t_lin = tpad.reshape(2 * vocab, EMBED)
    idxT = input.T.reshape(-1).astype(jnp.int32) * 2
    out5 = _sc_gather_t(idxT, t_lin).reshape(200, 8, 32, 8, 128)
    out = jnp.transpose(out5, (2, 4, 0, 1, 3)).reshape(4096, 200, EMBED)
    return out


# speedup vs baseline: 1.4599x; 1.4599x over previous
import functools

import jax
import jax.numpy as jnp
from jax import lax
from jax.experimental import pallas as pl
from jax.experimental.pallas import tpu as pltpu
from jax.experimental.pallas import tpu_sc as plsc

EMBED = 64
NUM_CORES = 2
NUM_SUBCORES = 16
NW = NUM_CORES * NUM_SUBCORES
BLK = 128
TPAD = 129


def _sc_gather_t(idx_flat, table2):
    b_total = idx_flat.shape[0]
    n_blocks = b_total // BLK
    blocks_per_w = n_blocks // NW
    b_per_w = b_total // NW
    mesh = plsc.VectorSubcoreMesh(core_axis_name="c", subcore_axis_name="s")

    @functools.partial(
        pl.kernel,
        out_type=jax.ShapeDtypeStruct((n_blocks * 8 * 8, 128), jnp.float32),
        mesh=mesh,
        scratch_types=[
            pltpu.VMEM((b_per_w,), jnp.int32),
            pltpu.VMEM((BLK, EMBED), jnp.float32),
            pltpu.VMEM((BLK, EMBED), jnp.float32),
            pltpu.VMEM((EMBED, TPAD), jnp.float32),
            pltpu.VMEM((EMBED, TPAD), jnp.float32),
            pltpu.SemaphoreType.DMA,
            pltpu.SemaphoreType.DMA,
            pltpu.SemaphoreType.DMA,
            pltpu.SemaphoreType.DMA,
        ],
        compiler_params=pltpu.CompilerParams(
            use_tc_tiling_on_sc=False, needs_layout_passes=False),
    )
    def k(idx_hbm, table_hbm, out_hbm, idx_v, rows0, rows1, t0, t1,
          gsem0, gsem1, osem0, osem1):
        wid = lax.axis_index("s") * NUM_CORES + lax.axis_index("c")
        base = wid * b_per_w
        rows = (rows0, rows1)
        tb = (t0, t1)
        gsem = (gsem0, gsem1)
        osem = (osem0, osem1)

        pltpu.sync_copy(idx_hbm.at[pl.ds(base, b_per_w)], idx_v)
        lanes = lax.iota(jnp.int32, 16)
        zeros16 = jnp.zeros((16,), jnp.int32)

        def gather(j, b):
            pltpu.async_copy(
                table_hbm.at[idx_v.at[pl.ds(j * BLK, BLK)]],
                rows[b], gsem[b])

        def stage(j, b):
            pltpu.make_async_copy(
                table_hbm.at[idx_v.at[pl.ds(0, BLK)]],
                rows[b], gsem[b]).wait()

            @plsc.parallel_loop(0, BLK, step=1, unroll=8)
            def _(bi):
                bvec = jnp.full((16,), bi, jnp.int32)
                for j16 in range(EMBED // 16):
                    vals = rows[b][bi, pl.ds(j16 * 16, 16)]
                    plsc.store_scatter(
                        tb[b], [j16 * 16 + lanes, bvec], vals)
            blk = wid * blocks_per_w + j
            h = blk // 32
            bc = blk - h * 32
            for er in range(8):
                pltpu.async_copy(
                    tb[b].at[pl.ds(er * 8, 8), pl.ds(0, 128)],
                    out_hbm.at[pl.ds(((h * 8 + er) * 32 + bc) * 8, 8)],
                    osem[b])

        def wait_writes(b):
            pltpu.make_async_copy(
                tb[b].at[pl.ds(0, EMBED), pl.ds(0, 128)],
                out_hbm.at[pl.ds(0, EMBED)], osem[b]).wait()

        gather(0, 0)

        def body(p, carry):
            for b in range(2):
                j = p * 2 + b
                if b == 0:
                    gather(j + 1, 1)
                else:
                    @pl.when(j + 1 < blocks_per_w)
                    def _():
                        gather(j + 1, 0)

                @pl.when(p >= 1)
                def _():
                    wait_writes(b)

                stage(j, b)
            return carry

        lax.fori_loop(0, blocks_per_w // 2, body, 0)
        wait_writes(0)
        wait_writes(1)

    return k(idx_flat, table2)


def kernel(input, table):
    vocab = table.shape[0]
    t_lin = jnp.pad(table, ((0, 0), (0, 64))).reshape(2 * vocab, EMBED)
    idxT = input.T.reshape(-1).astype(jnp.int32) * 2
    out5 = _sc_gather_t(idxT, t_lin).reshape(200, 8, 32, 8, 128)
    out = jnp.transpose(out5, (2, 4, 0, 1, 3)).reshape(4096, 200, EMBED)
    return out
